# Initial kernel scaffold; baseline (speedup 1.0000x reference)
import jax
import jax.numpy as jnp
from jax.experimental import pallas as pl


def kernel(x, W1, b1, W2, b2, ei0, ei1, ei2, nei0, nei1, nei2):
    raise NotImplementedError("write your pallas kernel here")


# speedup vs baseline: 1.7220x; 1.7220x over previous
import functools

import jax
import jax.numpy as jnp
from jax import lax
from jax.experimental import pallas as pl
from jax.experimental.pallas import tpu as pltpu
from jax.experimental.pallas import tpu_sc as plsc

N = 10000
E = 160000
R = 3
D = 128
DH = D // 2

NC = 2
NS = 16
NW = NC * NS

C = 128
E_PAD = 163840
ROWS2D = E_PAD // C
KPT_AGG = ROWS2D // NS
KPT_SCO = ROWS2D // NW
EPT_SCO = C * KPT_SCO

NPAD = 10240
NSTRIPE = NPAD // NS

_mesh = plsc.VectorSubcoreMesh(
    core_axis_name="c", subcore_axis_name="s", num_cores=NC, num_subcores=NS)


def _fill_rows(ref, nrows, ncols, value):
    def body(r, _):
        for j in range(ncols // 16):
            ref[r, pl.ds(j * 16, 16)] = jnp.full((16,), value, jnp.float32)
        return 0
    lax.fori_loop(0, nrows, body, 0)


def _agg_body(with_counts, h_hbm, s0, d0, s1, d1, s2, d2, *rest):
    if with_counts:
        (aggL_hbm, aggR_hbm, cnt_hbm, sidx, didx, rows, zbuf, zbuf16, ones,
         sem, acc, cacc) = rest
    else:
        (aggL_hbm, aggR_hbm, sidx, didx, rows, zbuf, sem, acc) = rest
        cacc = cnt_hbm = zbuf16 = ones = None
    cid = lax.axis_index("c")
    sid = lax.axis_index("s")

    _fill_rows(zbuf, 128, DH, 0.0)
    if with_counts:
        _fill_rows(zbuf16, 128, 16, 0.0)
        _fill_ones = functools.partial(_fill_rows, ones, C, 16, 1.0)
        _fill_ones()

    srcs = (s0, s1, s2)
    dsts = (d0, d1, d2)
    for r in range(R):
        for j in range(5):
            pltpu.sync_copy(zbuf, acc.at[pl.ds(sid * NSTRIPE + j * 128, 128)])
            if with_counts:
                @pl.when(cid == 0)
                def _():
                    pltpu.sync_copy(
                        zbuf16, cacc.at[pl.ds(sid * NSTRIPE + j * 128, 128)])
        plsc.subcore_barrier()

        pltpu.sync_copy(srcs[r].at[pl.ds(sid * KPT_AGG, KPT_AGG)], sidx)
        pltpu.sync_copy(dsts[r].at[pl.ds(sid * KPT_AGG, KPT_AGG)], didx)

        def chunk(k, _):
            pltpu.async_copy(h_hbm.at[cid].at[sidx.at[k]], rows, sem).wait()
            pltpu.sync_copy(rows, acc.at[didx.at[k]], add=True)
            if with_counts:
                @pl.when(cid == 0)
                def _():
                    pltpu.sync_copy(ones, cacc.at[didx.at[k]], add=True)
            return 0
        lax.fori_loop(0, KPT_AGG, chunk, 0)
        plsc.subcore_barrier()

        @pl.when(cid == 0)
        def _():
            pltpu.sync_copy(acc.at[pl.ds(sid * NSTRIPE, NSTRIPE)],
                            aggL_hbm.at[r, pl.ds(sid * NSTRIPE, NSTRIPE)])
            if with_counts:
                pltpu.sync_copy(cacc.at[pl.ds(sid * NSTRIPE, NSTRIPE)],
                                cnt_hbm.at[r, pl.ds(sid * NSTRIPE, NSTRIPE)])

        @pl.when(cid == 1)
        def _():
            pltpu.sync_copy(acc.at[pl.ds(sid * NSTRIPE, NSTRIPE)],
                            aggR_hbm.at[r, pl.ds(sid * NSTRIPE, NSTRIPE)])
        plsc.subcore_barrier()


def _make_agg_kernel(with_counts):
    out_type = [jax.ShapeDtypeStruct((R, NPAD, DH), jnp.float32),
                jax.ShapeDtypeStruct((R, NPAD, DH), jnp.float32)]
    scratch = [
        pltpu.VMEM((KPT_AGG, C), jnp.int32),
        pltpu.VMEM((KPT_AGG, C), jnp.int32),
        pltpu.VMEM((C, DH), jnp.float32),
        pltpu.VMEM((128, DH), jnp.float32),
    ]
    if with_counts:
        out_type.append(jax.ShapeDtypeStruct((R, NPAD, 16), jnp.float32))
        scratch += [
            pltpu.VMEM((128, 16), jnp.float32),
            pltpu.VMEM((C, 16), jnp.float32),
        ]
    scratch += [pltpu.SemaphoreType.DMA,
                pltpu.VMEM_SHARED((NPAD, DH), jnp.float32)]
    if with_counts:
        scratch.append(pltpu.VMEM_SHARED((NPAD, 16), jnp.float32))
    return pl.kernel(functools.partial(_agg_body, with_counts),
                     out_type=tuple(out_type), mesh=_mesh,
                     scratch_types=tuple(scratch),
                     compiler_params=pltpu.CompilerParams(
                         use_tc_tiling_on_sc=False))


def _scores_body(h_hbm, *args):
    idx_refs = args[:12]
    out_hbm, sidx, didx, hu, hv, dots, sem = args[12:]
    cid = lax.axis_index("c")
    sid = lax.axis_index("s")
    wid = cid * NS + sid

    for g in range(6):
        u2d = idx_refs[2 * g]
        v2d = idx_refs[2 * g + 1]
        pltpu.sync_copy(u2d.at[pl.ds(wid * KPT_SCO, KPT_SCO)], sidx)
        pltpu.sync_copy(v2d.at[pl.ds(wid * KPT_SCO, KPT_SCO)], didx)

        def chunk(k, _):
            cp_u = pltpu.async_copy(h_hbm.at[sidx.at[k]], hu, sem)
            cp_v = pltpu.async_copy(h_hbm.at[didx.at[k]], hv, sem)
            cp_u.wait()
            cp_v.wait()

            lane = lax.broadcasted_iota(jnp.int32, (16,), 0)

            def edge(e, _):
                a = hu[e, pl.ds(0, 16)] * hv[e, pl.ds(0, 16)]
                for j in range(1, D // 16):
                    a = a + hu[e, pl.ds(j * 16, 16)] * hv[e, pl.ds(j * 16, 16)]
                cs = plsc.cumsum(a)
                plsc.store_scatter(dots, [jnp.full((16,), e, jnp.int32)],
                                   cs, mask=lane == 15)
                return 0
            lax.fori_loop(0, C, edge, 0)

            for j in range(C // 16):
                v = dots[pl.ds(j * 16, 16)]
                dots[pl.ds(j * 16, 16)] = 1.0 / (1.0 + jnp.exp(-v))

            off = pl.multiple_of(wid * EPT_SCO + k * C, C)
            pltpu.sync_copy(dots, out_hbm.at[g, pl.ds(off, C)])
            return 0
        lax.fori_loop(0, KPT_SCO, chunk, 0)


_scores_kernel = pl.kernel(
    _scores_body,
    out_type=jax.ShapeDtypeStruct((6, E_PAD), jnp.float32),
    mesh=_mesh,
    scratch_types=(
        pltpu.VMEM((KPT_SCO, C), jnp.int32),
        pltpu.VMEM((KPT_SCO, C), jnp.int32),
        pltpu.VMEM((C, D), jnp.float32),
        pltpu.VMEM((C, D), jnp.float32),
        pltpu.VMEM((C,), jnp.float32),
        pltpu.SemaphoreType.DMA,
    ),
    compiler_params=pltpu.CompilerParams(needs_layout_passes=False))


def _dense_body(apply_act, split_out, aggL, aggR, cntp, w, b, *outs):
    nrows = aggL.shape[1]
    acc = jnp.zeros((nrows, D), jnp.float32)
    for r in range(R):
        cnt = cntp[r, :, 0]
        z = jax.lax.dot(aggL[r], w[r, :DH, :],
                        precision=jax.lax.Precision.HIGHEST,
                        preferred_element_type=jnp.float32)
        z = z + jax.lax.dot(aggR[r], w[r, DH:, :],
                            precision=jax.lax.Precision.HIGHEST,
                            preferred_element_type=jnp.float32)
        z = z + cnt[:, None] * b[r][None, :]
        acc = acc + z / jnp.maximum(cnt, 1.0)[:, None]
    if apply_act:
        acc = jnp.where(acc >= 0, acc, 0.01 * acc)
    if split_out:
        outs[0][0] = acc[:, :DH]
        outs[0][1] = acc[:, DH:]
    else:
        outs[0][...] = acc


def _dense_call(aggL, aggR, cntp, w, b, apply_act, split_out):
    blk = 1024
    grid = NPAD // blk
    if split_out:
        out_specs = pl.BlockSpec((NC, blk, DH), lambda i: (0, i, 0))
        out_shape = jax.ShapeDtypeStruct((NC, NPAD, DH), jnp.float32)
    else:
        out_specs = pl.BlockSpec((blk, D), lambda i: (i, 0))
        out_shape = jax.ShapeDtypeStruct((NPAD, D), jnp.float32)
    return pl.pallas_call(
        functools.partial(_dense_body, apply_act, split_out),
        grid=(grid,),
        in_specs=[
            pl.BlockSpec((R, blk, DH), lambda i: (0, i, 0)),
            pl.BlockSpec((R, blk, DH), lambda i: (0, i, 0)),
            pl.BlockSpec((R, blk, 16), lambda i: (0, i, 0)),
            pl.BlockSpec((R, D, D), lambda i: (0, 0, 0)),
            pl.BlockSpec((R, D), lambda i: (0, 0)),
        ],
        out_specs=out_specs,
        out_shape=out_shape,
    )(aggL, aggR, cntp, w, b)


def _pad_idx(vec, fill):
    pad = jnp.full((E_PAD - E,), fill, jnp.int32)
    return jnp.concatenate([vec, pad]).reshape(ROWS2D, C)


def kernel(x, W1, b1, W2, b2, ei0, ei1, ei2, nei0, nei1, nei2):
    pos = (ei0, ei1, ei2)
    agg_idx = []
    for ei in pos:
        agg_idx.append(_pad_idx(ei[0], 0))
        agg_idx.append(_pad_idx(ei[1], N))

    agg1 = _make_agg_kernel(True)
    agg2 = _make_agg_kernel(False)

    xstk = jnp.stack([x[:, :DH], x[:, DH:]])
    aggL1, aggR1, cntp = agg1(xstk, *agg_idx)
    h1stk = _dense_call(aggL1, aggR1, cntp, W1, b1, True, True)
    aggL2, aggR2 = agg2(h1stk, *agg_idx)
    h2 = _dense_call(aggL2, aggR2, cntp, W2, b2, False, False)

    sc_idx = []
    for ei in (ei0, ei1, ei2, nei0, nei1, nei2):
        sc_idx.append(_pad_idx(ei[0], 0))
        sc_idx.append(_pad_idx(ei[1], 0))
    scores = _scores_kernel(h2, *sc_idx)

    out_pos = scores[:R, :E].reshape(-1)
    out_neg = scores[R:, :E].reshape(-1)
    return (out_pos, out_neg)


# speedup vs baseline: 2.0380x; 1.1835x over previous
import functools

import jax
import jax.numpy as jnp
from jax import lax
from jax.experimental import pallas as pl
from jax.experimental.pallas import tpu as pltpu
from jax.experimental.pallas import tpu_sc as plsc

N = 10000
E = 160000
R = 3
D = 128
DH = D // 2

NC = 2
NS = 16
NW = NC * NS

C = 128
E_PAD = 163840
ROWS2D = E_PAD // C
KPT_AGG = ROWS2D // NS
KPT_SCO = ROWS2D // NW
EPT_SCO = C * KPT_SCO

NPAD = 10240
NSTRIPE = NPAD // NS

_mesh = plsc.VectorSubcoreMesh(
    core_axis_name="c", subcore_axis_name="s", num_cores=NC, num_subcores=NS)


def _fill_rows(ref, nrows, ncols, value):
    def body(r, _):
        for j in range(ncols // 16):
            ref[r, pl.ds(j * 16, 16)] = jnp.full((16,), value, jnp.float32)
        return 0
    lax.fori_loop(0, nrows, body, 0)


def _agg_body(with_counts, h_hbm, s0, d0, s1, d1, s2, d2, *rest):
    if with_counts:
        (aggL_hbm, aggR_hbm, cnt_hbm, sidx, didx, rows0, rows1, zbuf, zbuf16,
         ones, gsem0, gsem1, acc, cacc) = rest
    else:
        (aggL_hbm, aggR_hbm, sidx, didx, rows0, rows1, zbuf, gsem0, gsem1,
         acc) = rest
        cacc = cnt_hbm = zbuf16 = ones = None
    cid = lax.axis_index("c")
    sid = lax.axis_index("s")

    _fill_rows(zbuf, 128, DH, 0.0)
    if with_counts:
        _fill_rows(zbuf16, 128, 16, 0.0)
        _fill_ones = functools.partial(_fill_rows, ones, C, 16, 1.0)
        _fill_ones()

    srcs = (s0, s1, s2)
    dsts = (d0, d1, d2)
    for r in range(R):
        for j in range(5):
            pltpu.sync_copy(zbuf, acc.at[pl.ds(sid * NSTRIPE + j * 128, 128)])
            if with_counts:
                @pl.when(cid == 0)
                def _():
                    pltpu.sync_copy(
                        zbuf16, cacc.at[pl.ds(sid * NSTRIPE + j * 128, 128)])
        plsc.subcore_barrier()

        pltpu.sync_copy(srcs[r].at[pl.ds(sid * KPT_AGG, KPT_AGG)], sidx)
        pltpu.sync_copy(dsts[r].at[pl.ds(sid * KPT_AGG, KPT_AGG)], didx)

        rows = (rows0, rows1)
        gsem = (gsem0, gsem1)
        pltpu.async_copy(h_hbm.at[cid].at[sidx.at[0]], rows0, gsem0)
        pltpu.async_copy(h_hbm.at[cid].at[sidx.at[1]], rows1, gsem1)

        def chunk_pair(k2, _):
            k = k2 * 2
            for b in range(2):
                pltpu.make_async_copy(
                    h_hbm.at[cid].at[sidx.at[k + b]], rows[b], gsem[b]).wait()
                pltpu.sync_copy(rows[b], acc.at[didx.at[k + b]], add=True)
                if with_counts:
                    @pl.when(cid == 0)
                    def _():
                        pltpu.sync_copy(ones, cacc.at[didx.at[k + b]],
                                        add=True)

                @pl.when(k + b + 2 < KPT_AGG)
                def _():
                    pltpu.async_copy(
                        h_hbm.at[cid].at[sidx.at[k + b + 2]], rows[b],
                        gsem[b])
            return 0
        lax.fori_loop(0, KPT_AGG // 2, chunk_pair, 0)
        plsc.subcore_barrier()

        @pl.when(cid == 0)
        def _():
            pltpu.sync_copy(acc.at[pl.ds(sid * NSTRIPE, NSTRIPE)],
                            aggL_hbm.at[r, pl.ds(sid * NSTRIPE, NSTRIPE)])
            if with_counts:
                pltpu.sync_copy(cacc.at[pl.ds(sid * NSTRIPE, NSTRIPE)],
                                cnt_hbm.at[r, pl.ds(sid * NSTRIPE, NSTRIPE)])

        @pl.when(cid == 1)
        def _():
            pltpu.sync_copy(acc.at[pl.ds(sid * NSTRIPE, NSTRIPE)],
                            aggR_hbm.at[r, pl.ds(sid * NSTRIPE, NSTRIPE)])
        plsc.subcore_barrier()


def _make_agg_kernel(with_counts):
    out_type = [jax.ShapeDtypeStruct((R, NPAD, DH), jnp.float32),
                jax.ShapeDtypeStruct((R, NPAD, DH), jnp.float32)]
    scratch = [
        pltpu.VMEM((KPT_AGG, C), jnp.int32),
        pltpu.VMEM((KPT_AGG, C), jnp.int32),
        pltpu.VMEM((C, DH), jnp.float32),
        pltpu.VMEM((C, DH), jnp.float32),
        pltpu.VMEM((128, DH), jnp.float32),
    ]
    if with_counts:
        out_type.append(jax.ShapeDtypeStruct((R, NPAD, 16), jnp.float32))
        scratch += [
            pltpu.VMEM((128, 16), jnp.float32),
            pltpu.VMEM((C, 16), jnp.float32),
        ]
    scratch += [pltpu.SemaphoreType.DMA, pltpu.SemaphoreType.DMA,
                pltpu.VMEM_SHARED((NPAD, DH), jnp.float32)]
    if with_counts:
        scratch.append(pltpu.VMEM_SHARED((NPAD, 16), jnp.float32))
    return pl.kernel(functools.partial(_agg_body, with_counts),
                     out_type=tuple(out_type), mesh=_mesh,
                     scratch_types=tuple(scratch),
                     compiler_params=pltpu.CompilerParams(
                         use_tc_tiling_on_sc=False))


def _scores_body(h_hbm, *args):
    idx_refs = args[:12]
    (out_hbm, sidx, didx, hu0, hv0, hu1, hv1, tbuf, dots,
     gsem0, gsem1) = args[12:]
    cid = lax.axis_index("c")
    sid = lax.axis_index("s")
    wid = cid * NS + sid

    hu = (hu0, hu1)
    hv = (hv0, hv1)
    gsem = (gsem0, gsem1)
    lane = lax.broadcasted_iota(jnp.int32, (16,), 0)

    def issue(k, b):
        pltpu.async_copy(h_hbm.at[sidx.at[k]], hu[b], gsem[b])
        pltpu.async_copy(h_hbm.at[didx.at[k]], hv[b], gsem[b])

    def drain(k, b):
        pltpu.make_async_copy(h_hbm.at[sidx.at[k]], hu[b], gsem[b]).wait()
        pltpu.make_async_copy(h_hbm.at[didx.at[k]], hv[b], gsem[b]).wait()

    def compute(k, b, g):
        def group(gi, _):
            e0 = gi * 16
            for i in range(16):
                e = e0 + i
                a = hu[b][e, pl.ds(0, 16)] * hv[b][e, pl.ds(0, 16)]
                for j in range(1, D // 16):
                    a = a + (hu[b][e, pl.ds(j * 16, 16)]
                             * hv[b][e, pl.ds(j * 16, 16)])
                tbuf[i, pl.ds(0, 16)] = a
            s = plsc.load_gather(tbuf, [lane, jnp.zeros((16,), jnp.int32)])
            for cc in range(1, 16):
                s = s + plsc.load_gather(
                    tbuf, [lane, jnp.full((16,), cc, jnp.int32)])
            s = 1.0 / (1.0 + jnp.exp(-s))
            dots[pl.ds(e0, 16)] = s
            return 0
        lax.fori_loop(0, C // 16, group, 0)
        off = pl.multiple_of(wid * EPT_SCO, C) + pl.multiple_of(k * C, C)
        pltpu.sync_copy(dots, out_hbm.at[g, pl.ds(off, C)])

    for g in range(6):
        u2d = idx_refs[2 * g]
        v2d = idx_refs[2 * g + 1]
        pltpu.sync_copy(u2d.at[pl.ds(wid * KPT_SCO, KPT_SCO)], sidx)
        pltpu.sync_copy(v2d.at[pl.ds(wid * KPT_SCO, KPT_SCO)], didx)

        issue(0, 0)
        issue(1, 1)

        def chunk_pair(k2, _):
            k = k2 * 2
            for b in range(2):
                drain(k + b, b)
                compute(k + b, b, g)

                @pl.when(k + b + 2 < KPT_SCO)
                def _():
                    issue(k + b + 2, b)
            return 0
        lax.fori_loop(0, KPT_SCO // 2, chunk_pair, 0)


_scores_kernel = pl.kernel(
    _scores_body,
    out_type=jax.ShapeDtypeStruct((6, E_PAD), jnp.float32),
    mesh=_mesh,
    scratch_types=(
        pltpu.VMEM((KPT_SCO, C), jnp.int32),
        pltpu.VMEM((KPT_SCO, C), jnp.int32),
        pltpu.VMEM((C, D), jnp.float32),
        pltpu.VMEM((C, D), jnp.float32),
        pltpu.VMEM((C, D), jnp.float32),
        pltpu.VMEM((C, D), jnp.float32),
        pltpu.VMEM((16, 17), jnp.float32),
        pltpu.VMEM((C,), jnp.float32),
        pltpu.SemaphoreType.DMA,
        pltpu.SemaphoreType.DMA,
    ),
    compiler_params=pltpu.CompilerParams(needs_layout_passes=False))


def _dense_body(apply_act, split_out, aggL, aggR, cntp, w, b, *outs):
    nrows = aggL.shape[1]
    acc = jnp.zeros((nrows, D), jnp.float32)
    for r in range(R):
        cnt = cntp[r, :, 0]
        z = jax.lax.dot(aggL[r], w[r, :DH, :],
                        precision=jax.lax.Precision.HIGHEST,
                        preferred_element_type=jnp.float32)
        z = z + jax.lax.dot(aggR[r], w[r, DH:, :],
                            precision=jax.lax.Precision.HIGHEST,
                            preferred_element_type=jnp.float32)
        z = z + cnt[:, None] * b[r][None, :]
        acc = acc + z / jnp.maximum(cnt, 1.0)[:, None]
    if apply_act:
        acc = jnp.where(acc >= 0, acc, 0.01 * acc)
    if split_out:
        outs[0][0] = acc[:, :DH]
        outs[0][1] = acc[:, DH:]
    else:
        outs[0][...] = acc


def _dense_call(aggL, aggR, cntp, w, b, apply_act, split_out):
    blk = 1024
    grid = NPAD // blk
    if split_out:
        out_specs = pl.BlockSpec((NC, blk, DH), lambda i: (0, i, 0))
        out_shape = jax.ShapeDtypeStruct((NC, NPAD, DH), jnp.float32)
    else:
        out_specs = pl.BlockSpec((blk, D), lambda i: (i, 0))
        out_shape = jax.ShapeDtypeStruct((NPAD, D), jnp.float32)
    return pl.pallas_call(
        functools.partial(_dense_body, apply_act, split_out),
        grid=(grid,),
        in_specs=[
            pl.BlockSpec((R, blk, DH), lambda i: (0, i, 0)),
            pl.BlockSpec((R, blk, DH), lambda i: (0, i, 0)),
            pl.BlockSpec((R, blk, 16), lambda i: (0, i, 0)),
            pl.BlockSpec((R, D, D), lambda i: (0, 0, 0)),
            pl.BlockSpec((R, D), lambda i: (0, 0)),
        ],
        out_specs=out_specs,
        out_shape=out_shape,
    )(aggL, aggR, cntp, w, b)


def _pad_idx(vec, fill):
    pad = jnp.full((E_PAD - E,), fill, jnp.int32)
    return jnp.concatenate([vec, pad]).reshape(ROWS2D, C)


def kernel(x, W1, b1, W2, b2, ei0, ei1, ei2, nei0, nei1, nei2):
    pos = (ei0, ei1, ei2)
    agg_idx = []
    for ei in pos:
        agg_idx.append(_pad_idx(ei[0], 0))
        agg_idx.append(_pad_idx(ei[1], N))

    agg1 = _make_agg_kernel(True)
    agg2 = _make_agg_kernel(False)

    xstk = jnp.stack([x[:, :DH], x[:, DH:]])
    aggL1, aggR1, cntp = agg1(xstk, *agg_idx)
    h1stk = _dense_call(aggL1, aggR1, cntp, W1, b1, True, True)
    aggL2, aggR2 = agg2(h1stk, *agg_idx)
    h2 = _dense_call(aggL2, aggR2, cntp, W2, b2, False, False)

    sc_idx = []
    for ei in (ei0, ei1, ei2, nei0, nei1, nei2):
        sc_idx.append(_pad_idx(ei[0], 0))
        sc_idx.append(_pad_idx(ei[1], 0))
    scores = _scores_kernel(h2, *sc_idx)

    out_pos = scores[:R, :E].reshape(-1)
    out_neg = scores[R:, :E].reshape(-1)
    return (out_pos, out_neg)


# speedup vs baseline: 6.8494x; 3.3608x over previous
import functools

import jax
import jax.numpy as jnp
from jax import lax
from jax.experimental import pallas as pl
from jax.experimental.pallas import tpu as pltpu
from jax.experimental.pallas import tpu_sc as plsc

N = 10000
E = 160000
R = 3
D = 128
DH = D // 2

NC = 2
NS = 16
NW = NC * NS

C = 128
E_PAD = 163840
ROWS2D = E_PAD // C
KPT_AGG = ROWS2D // NS
KPT_SCO = ROWS2D // NW
EPT_SCO = C * KPT_SCO

NPAD = 10240
NSTRIPE = NPAD // NS

_mesh = plsc.VectorSubcoreMesh(
    core_axis_name="c", subcore_axis_name="s", num_cores=NC, num_subcores=NS)


def _fill_rows(ref, nrows, ncols, value):
    def body(r, _):
        for j in range(ncols // 16):
            ref[r, pl.ds(j * 16, 16)] = jnp.full((16,), value, jnp.float32)
        return 0
    lax.fori_loop(0, nrows, body, 0)


def _agg_body(with_counts, h_hbm, s0, d0, s1, d1, s2, d2, *rest):
    if with_counts:
        (aggL_hbm, aggR_hbm, cnt_hbm, sidx, didx, rows0, rows1, zbuf, zbuf16,
         ones, gsem0, gsem1, acc, cacc) = rest
    else:
        (aggL_hbm, aggR_hbm, sidx, didx, rows0, rows1, zbuf, gsem0, gsem1,
         acc) = rest
        cacc = cnt_hbm = zbuf16 = ones = None
    cid = lax.axis_index("c")
    sid = lax.axis_index("s")

    _fill_rows(zbuf, 128, DH, 0.0)
    if with_counts:
        _fill_rows(zbuf16, 128, 16, 0.0)
        _fill_ones = functools.partial(_fill_rows, ones, C, 16, 1.0)
        _fill_ones()

    srcs = (s0, s1, s2)
    dsts = (d0, d1, d2)
    for r in range(R):
        for j in range(5):
            pltpu.sync_copy(zbuf, acc.at[pl.ds(sid * NSTRIPE + j * 128, 128)])
            if with_counts:
                @pl.when(cid == 0)
                def _():
                    pltpu.sync_copy(
                        zbuf16, cacc.at[pl.ds(sid * NSTRIPE + j * 128, 128)])
        plsc.subcore_barrier()

        pltpu.sync_copy(srcs[r].at[pl.ds(sid * KPT_AGG, KPT_AGG)], sidx)
        pltpu.sync_copy(dsts[r].at[pl.ds(sid * KPT_AGG, KPT_AGG)], didx)

        rows = (rows0, rows1)
        gsem = (gsem0, gsem1)
        pltpu.async_copy(h_hbm.at[cid].at[sidx.at[0]], rows0, gsem0)
        pltpu.async_copy(h_hbm.at[cid].at[sidx.at[1]], rows1, gsem1)

        def chunk_pair(k2, _):
            k = k2 * 2
            for b in range(2):
                pltpu.make_async_copy(
                    h_hbm.at[cid].at[sidx.at[k + b]], rows[b], gsem[b]).wait()
                pltpu.sync_copy(rows[b], acc.at[didx.at[k + b]], add=True)
                if with_counts:
                    @pl.when(cid == 0)
                    def _():
                        pltpu.sync_copy(ones, cacc.at[didx.at[k + b]],
                                        add=True)

                @pl.when(k + b + 2 < KPT_AGG)
                def _():
                    pltpu.async_copy(
                        h_hbm.at[cid].at[sidx.at[k + b + 2]], rows[b],
                        gsem[b])
            return 0
        lax.fori_loop(0, KPT_AGG // 2, chunk_pair, 0)
        plsc.subcore_barrier()

        @pl.when(cid == 0)
        def _():
            pltpu.sync_copy(acc.at[pl.ds(sid * NSTRIPE, NSTRIPE)],
                            aggL_hbm.at[r, pl.ds(sid * NSTRIPE, NSTRIPE)])
            if with_counts:
                pltpu.sync_copy(cacc.at[pl.ds(sid * NSTRIPE, NSTRIPE)],
                                cnt_hbm.at[r, pl.ds(sid * NSTRIPE, NSTRIPE)])

        @pl.when(cid == 1)
        def _():
            pltpu.sync_copy(acc.at[pl.ds(sid * NSTRIPE, NSTRIPE)],
                            aggR_hbm.at[r, pl.ds(sid * NSTRIPE, NSTRIPE)])
        plsc.subcore_barrier()


def _make_agg_kernel(with_counts):
    out_type = [jax.ShapeDtypeStruct((R, NPAD, DH), jnp.float32),
                jax.ShapeDtypeStruct((R, NPAD, DH), jnp.float32)]
    scratch = [
        pltpu.VMEM((KPT_AGG, C), jnp.int32),
        pltpu.VMEM((KPT_AGG, C), jnp.int32),
        pltpu.VMEM((C, DH), jnp.float32),
        pltpu.VMEM((C, DH), jnp.float32),
        pltpu.VMEM((128, DH), jnp.float32),
    ]
    if with_counts:
        out_type.append(jax.ShapeDtypeStruct((R, NPAD, 16), jnp.float32))
        scratch += [
            pltpu.VMEM((128, 16), jnp.float32),
            pltpu.VMEM((C, 16), jnp.float32),
        ]
    scratch += [pltpu.SemaphoreType.DMA, pltpu.SemaphoreType.DMA,
                pltpu.VMEM_SHARED((NPAD, DH), jnp.float32)]
    if with_counts:
        scratch.append(pltpu.VMEM_SHARED((NPAD, 16), jnp.float32))
    return pl.kernel(functools.partial(_agg_body, with_counts),
                     out_type=tuple(out_type), mesh=_mesh,
                     scratch_types=tuple(scratch),
                     compiler_params=pltpu.CompilerParams(
                         use_tc_tiling_on_sc=False))


def _scores_body(h_hbm, *args):
    idx_refs = args[:12]
    (out_hbm, sidx, didx, hu0, hv0, hu1, hv1, tbuf, dots,
     gsem0, gsem1) = args[12:]
    cid = lax.axis_index("c")
    sid = lax.axis_index("s")
    wid = cid * NS + sid

    hu = (hu0, hu1)
    hv = (hv0, hv1)
    gsem = (gsem0, gsem1)
    lane = lax.broadcasted_iota(jnp.int32, (16,), 0)

    def issue(k, b):
        pltpu.async_copy(h_hbm.at[sidx.at[k]], hu[b], gsem[b])
        pltpu.async_copy(h_hbm.at[didx.at[k]], hv[b], gsem[b])

    def drain(k, b):
        pltpu.make_async_copy(h_hbm.at[sidx.at[k]], hu[b], gsem[b]).wait()
        pltpu.make_async_copy(h_hbm.at[didx.at[k]], hv[b], gsem[b]).wait()

    def compute(k, b, g):
        def group(gi, _):
            e0 = gi * 16
            for i in range(16):
                e = e0 + i
                a = hu[b][e, pl.ds(0, 16)] * hv[b][e, pl.ds(0, 16)]
                for j in range(1, D // 16):
                    a = a + (hu[b][e, pl.ds(j * 16, 16)]
                             * hv[b][e, pl.ds(j * 16, 16)])
                tbuf[i, pl.ds(0, 16)] = a
            s = plsc.load_gather(tbuf, [lane, jnp.zeros((16,), jnp.int32)])
            for cc in range(1, 16):
                s = s + plsc.load_gather(
                    tbuf, [lane, jnp.full((16,), cc, jnp.int32)])
            s = 1.0 / (1.0 + jnp.exp(-s))
            dots[pl.ds(e0, 16)] = s
            return 0
        lax.fori_loop(0, C // 16, group, 0)
        off = pl.multiple_of(wid * EPT_SCO, C) + pl.multiple_of(k * C, C)
        pltpu.sync_copy(dots, out_hbm.at[g, pl.ds(off, C)])

    for g in range(6):
        u2d = idx_refs[2 * g]
        v2d = idx_refs[2 * g + 1]
        pltpu.sync_copy(u2d.at[pl.ds(wid * KPT_SCO, KPT_SCO)], sidx)
        pltpu.sync_copy(v2d.at[pl.ds(wid * KPT_SCO, KPT_SCO)], didx)

        issue(0, 0)
        issue(1, 1)

        def chunk_pair(k2, _):
            k = k2 * 2
            for b in range(2):
                drain(k + b, b)
                compute(k + b, b, g)

                @pl.when(k + b + 2 < KPT_SCO)
                def _():
                    issue(k + b + 2, b)
            return 0
        lax.fori_loop(0, KPT_SCO // 2, chunk_pair, 0)


_scores_kernel = pl.kernel(
    _scores_body,
    out_type=jax.ShapeDtypeStruct((6, E_PAD), jnp.float32),
    mesh=_mesh,
    scratch_types=(
        pltpu.VMEM((KPT_SCO, C), jnp.int32),
        pltpu.VMEM((KPT_SCO, C), jnp.int32),
        pltpu.VMEM((C, D), jnp.float32),
        pltpu.VMEM((C, D), jnp.float32),
        pltpu.VMEM((C, D), jnp.float32),
        pltpu.VMEM((C, D), jnp.float32),
        pltpu.VMEM((16, 17), jnp.float32),
        pltpu.VMEM((C,), jnp.float32),
        pltpu.SemaphoreType.DMA,
        pltpu.SemaphoreType.DMA,
    ),
    compiler_params=pltpu.CompilerParams(needs_layout_passes=False))


def _dense_body(apply_act, split_out, aggL, aggR, cntp, w, b, *outs):
    nrows = aggL.shape[1]
    acc = jnp.zeros((nrows, D), jnp.float32)
    for r in range(R):
        cnt = cntp[r, :, 0]
        z = jax.lax.dot(aggL[r], w[r, :DH, :],
                        precision=jax.lax.Precision.HIGHEST,
                        preferred_element_type=jnp.float32)
        z = z + jax.lax.dot(aggR[r], w[r, DH:, :],
                            precision=jax.lax.Precision.HIGHEST,
                            preferred_element_type=jnp.float32)
        z = z + cnt[:, None] * b[r][None, :]
        acc = acc + z / jnp.maximum(cnt, 1.0)[:, None]
    if apply_act:
        acc = jnp.where(acc >= 0, acc, 0.01 * acc)
    if split_out:
        outs[0][0] = acc[:, :DH]
        outs[0][1] = acc[:, DH:]
    else:
        outs[0][...] = acc


def _dense_call(aggL, aggR, cntp, w, b, apply_act, split_out):
    blk = 1024
    grid = NPAD // blk
    if split_out:
        out_specs = pl.BlockSpec((NC, blk, DH), lambda i: (0, i, 0))
        out_shape = jax.ShapeDtypeStruct((NC, NPAD, DH), jnp.float32)
    else:
        out_specs = pl.BlockSpec((blk, D), lambda i: (i, 0))
        out_shape = jax.ShapeDtypeStruct((NPAD, D), jnp.float32)
    return pl.pallas_call(
        functools.partial(_dense_body, apply_act, split_out),
        grid=(grid,),
        in_specs=[
            pl.BlockSpec((R, blk, DH), lambda i: (0, i, 0)),
            pl.BlockSpec((R, blk, DH), lambda i: (0, i, 0)),
            pl.BlockSpec((R, blk, 16), lambda i: (0, i, 0)),
            pl.BlockSpec((R, D, D), lambda i: (0, 0, 0)),
            pl.BlockSpec((R, D), lambda i: (0, 0)),
        ],
        out_specs=out_specs,
        out_shape=out_shape,
    )(aggL, aggR, cntp, w, b)


def _pad_idx(vec, fill):
    return jnp.concatenate([vec, fill]).reshape(ROWS2D, C)


def kernel(x, W1, b1, W2, b2, ei0, ei1, ei2, nei0, nei1, nei2):
    pos = (ei0, ei1, ei2)
    npad = E_PAD - E
    src_fill = jnp.arange(npad, dtype=jnp.int32) % N
    dst_fill = N + (jnp.arange(npad, dtype=jnp.int32) % (NPAD - N))
    agg_idx = []
    for ei in pos:
        agg_idx.append(_pad_idx(ei[0], src_fill))
        agg_idx.append(_pad_idx(ei[1], dst_fill))

    agg1 = _make_agg_kernel(True)
    agg2 = _make_agg_kernel(False)

    xstk = jnp.stack([x[:, :DH], x[:, DH:]])
    aggL1, aggR1, cntp = agg1(xstk, *agg_idx)
    h1stk = _dense_call(aggL1, aggR1, cntp, W1, b1, True, True)
    aggL2, aggR2 = agg2(h1stk, *agg_idx)
    h2 = _dense_call(aggL2, aggR2, cntp, W2, b2, False, False)

    sc_idx = []
    for ei in (ei0, ei1, ei2, nei0, nei1, nei2):
        sc_idx.append(_pad_idx(ei[0], src_fill))
        sc_idx.append(_pad_idx(ei[1], src_fill))
    scores = _scores_kernel(h2, *sc_idx)

    out_pos = scores[:R, :E].reshape(-1)
    out_neg = scores[R:, :E].reshape(-1)
    return (out_pos, out_neg)


# speedup vs baseline: 7.3837x; 1.0780x over previous
import functools

import jax
import jax.numpy as jnp
from jax import lax
from jax.experimental import pallas as pl
from jax.experimental.pallas import tpu as pltpu
from jax.experimental.pallas import tpu_sc as plsc

N = 10000
E = 160000
R = 3
D = 128
DH = D // 2

NC = 2
NS = 16
NW = NC * NS

C = 128
E_PAD = 163840
ROWS2D = E_PAD // C
KPT_AGG = ROWS2D // NS
KPT_SCO = ROWS2D // NW
EPT_SCO = C * KPT_SCO

NPAD = 10240
NSTRIPE = NPAD // NS

_mesh = plsc.VectorSubcoreMesh(
    core_axis_name="c", subcore_axis_name="s", num_cores=NC, num_subcores=NS)


def _fill_rows(ref, nrows, ncols, value):
    def body(r, _):
        for j in range(ncols // 16):
            ref[r, pl.ds(j * 16, 16)] = jnp.full((16,), value, jnp.float32)
        return 0
    lax.fori_loop(0, nrows, body, 0)


def _agg_body(with_counts, h_hbm, s0, d0, s1, d1, s2, d2, *rest):
    if with_counts:
        (aggL_hbm, aggR_hbm, cnt_hbm, sidx, didx, r0, r1, r2, r3, zbuf,
         zbuf16, ones, g0, g1, g2, g3, s0_, s1_, s2_, s3_, acc, cacc) = rest
    else:
        (aggL_hbm, aggR_hbm, sidx, didx, r0, r1, r2, r3, zbuf,
         g0, g1, g2, g3, s0_, s1_, s2_, s3_, acc) = rest
        cacc = cnt_hbm = zbuf16 = ones = None
    rows = (r0, r1, r2, r3)
    gsem = (g0, g1, g2, g3)
    ssem = (s0_, s1_, s2_, s3_)
    cid = lax.axis_index("c")
    sid = lax.axis_index("s")

    _fill_rows(zbuf, 128, DH, 0.0)
    if with_counts:
        _fill_rows(zbuf16, 128, 16, 0.0)
        _fill_ones = functools.partial(_fill_rows, ones, C, 16, 1.0)
        _fill_ones()

    srcs = (s0, s1, s2)
    dsts = (d0, d1, d2)
    for r in range(R):
        for j in range(5):
            pltpu.sync_copy(zbuf, acc.at[pl.ds(sid * NSTRIPE + j * 128, 128)])
            if with_counts:
                @pl.when(cid == 0)
                def _():
                    pltpu.sync_copy(
                        zbuf16, cacc.at[pl.ds(sid * NSTRIPE + j * 128, 128)])
        plsc.subcore_barrier()

        pltpu.sync_copy(srcs[r].at[pl.ds(sid * KPT_AGG, KPT_AGG)], sidx)
        pltpu.sync_copy(dsts[r].at[pl.ds(sid * KPT_AGG, KPT_AGG)], didx)

        NB = 4
        for b in range(NB):
            pltpu.async_copy(h_hbm.at[cid].at[sidx.at[b]], rows[b], gsem[b])

        def chunk_quad(k4, _):
            k = k4 * NB
            for b in range(NB):
                pltpu.make_async_copy(
                    h_hbm.at[cid].at[sidx.at[k + b]], rows[b], gsem[b]).wait()
                cp = pltpu.async_copy(rows[b], acc.at[didx.at[k + b]],
                                      ssem[b], add=True)
                if with_counts:
                    @pl.when(cid == 0)
                    def _():
                        pltpu.sync_copy(ones, cacc.at[didx.at[k + b]],
                                        add=True)
                cp.wait()

                @pl.when(k + b + NB < KPT_AGG)
                def _():
                    pltpu.async_copy(
                        h_hbm.at[cid].at[sidx.at[k + b + NB]], rows[b],
                        gsem[b])
            return 0
        lax.fori_loop(0, KPT_AGG // NB, chunk_quad, 0)
        plsc.subcore_barrier()

        @pl.when(cid == 0)
        def _():
            pltpu.sync_copy(acc.at[pl.ds(sid * NSTRIPE, NSTRIPE)],
                            aggL_hbm.at[r, pl.ds(sid * NSTRIPE, NSTRIPE)])
            if with_counts:
                pltpu.sync_copy(cacc.at[pl.ds(sid * NSTRIPE, NSTRIPE)],
                                cnt_hbm.at[r, pl.ds(sid * NSTRIPE, NSTRIPE)])

        @pl.when(cid == 1)
        def _():
            pltpu.sync_copy(acc.at[pl.ds(sid * NSTRIPE, NSTRIPE)],
                            aggR_hbm.at[r, pl.ds(sid * NSTRIPE, NSTRIPE)])
        plsc.subcore_barrier()


def _make_agg_kernel(with_counts):
    out_type = [jax.ShapeDtypeStruct((R, NPAD, DH), jnp.float32),
                jax.ShapeDtypeStruct((R, NPAD, DH), jnp.float32)]
    scratch = [
        pltpu.VMEM((KPT_AGG, C), jnp.int32),
        pltpu.VMEM((KPT_AGG, C), jnp.int32),
        pltpu.VMEM((C, DH), jnp.float32),
        pltpu.VMEM((C, DH), jnp.float32),
        pltpu.VMEM((C, DH), jnp.float32),
        pltpu.VMEM((C, DH), jnp.float32),
        pltpu.VMEM((128, DH), jnp.float32),
    ]
    if with_counts:
        out_type.append(jax.ShapeDtypeStruct((R, NPAD, 16), jnp.float32))
        scratch += [
            pltpu.VMEM((128, 16), jnp.float32),
            pltpu.VMEM((C, 16), jnp.float32),
        ]
    scratch += [pltpu.SemaphoreType.DMA] * 8 + [
                pltpu.VMEM_SHARED((NPAD, DH), jnp.float32)]
    if with_counts:
        scratch.append(pltpu.VMEM_SHARED((NPAD, 16), jnp.float32))
    return pl.kernel(functools.partial(_agg_body, with_counts),
                     out_type=tuple(out_type), mesh=_mesh,
                     scratch_types=tuple(scratch),
                     compiler_params=pltpu.CompilerParams(
                         use_tc_tiling_on_sc=False))


KPT_ALL = 6 * ROWS2D // NW


def _scores_body(h_hbm, uall, vall, out_hbm, *rest):
    (sidx, didx, hu0, hv0, hu1, hv1, tbuf, dots, gsem0, gsem1) = rest
    cid = lax.axis_index("c")
    sid = lax.axis_index("s")
    wid = cid * NS + sid

    hu = (hu0, hu1)
    hv = (hv0, hv1)
    gsem = (gsem0, gsem1)
    lane = lax.broadcasted_iota(jnp.int32, (16,), 0)

    pltpu.sync_copy(uall.at[pl.ds(wid * KPT_ALL, KPT_ALL)], sidx)
    pltpu.sync_copy(vall.at[pl.ds(wid * KPT_ALL, KPT_ALL)], didx)

    def issue(k, b):
        pltpu.async_copy(h_hbm.at[sidx.at[k]], hu[b], gsem[b])
        pltpu.async_copy(h_hbm.at[didx.at[k]], hv[b], gsem[b])

    def drain(k, b):
        pltpu.make_async_copy(h_hbm.at[sidx.at[k]], hu[b], gsem[b]).wait()
        pltpu.make_async_copy(h_hbm.at[didx.at[k]], hv[b], gsem[b]).wait()

    def compute(k, b):
        def group(gi, _):
            e0 = gi * 16
            for i in range(16):
                e = e0 + i
                a = hu[b][e, pl.ds(0, 16)] * hv[b][e, pl.ds(0, 16)]
                for j in range(1, D // 16):
                    a = a + (hu[b][e, pl.ds(j * 16, 16)]
                             * hv[b][e, pl.ds(j * 16, 16)])
                tbuf[i, pl.ds(0, 16)] = a
            s = plsc.load_gather(tbuf, [lane, jnp.zeros((16,), jnp.int32)])
            for cc in range(1, 16):
                s = s + plsc.load_gather(
                    tbuf, [lane, jnp.full((16,), cc, jnp.int32)])
            s = 1.0 / (1.0 + jnp.exp(-s))
            dots[pl.ds(e0, 16)] = s
            return 0
        lax.fori_loop(0, C // 16, group, 0)
        off = pl.multiple_of(wid * KPT_ALL * C, C) + pl.multiple_of(k * C, C)
        pltpu.sync_copy(dots, out_hbm.at[pl.ds(off, C)])

    issue(0, 0)
    issue(1, 1)

    def chunk_pair(k2, _):
        k = k2 * 2
        for b in range(2):
            drain(k + b, b)
            compute(k + b, b)

            @pl.when(k + b + 2 < KPT_ALL)
            def _():
                issue(k + b + 2, b)
        return 0
    lax.fori_loop(0, KPT_ALL // 2, chunk_pair, 0)


_scores_kernel = pl.kernel(
    _scores_body,
    out_type=jax.ShapeDtypeStruct((6 * E_PAD,), jnp.float32),
    mesh=_mesh,
    scratch_types=(
        pltpu.VMEM((KPT_ALL, C), jnp.int32),
        pltpu.VMEM((KPT_ALL, C), jnp.int32),
        pltpu.VMEM((C, D), jnp.float32),
        pltpu.VMEM((C, D), jnp.float32),
        pltpu.VMEM((C, D), jnp.float32),
        pltpu.VMEM((C, D), jnp.float32),
        pltpu.VMEM((16, 17), jnp.float32),
        pltpu.VMEM((C,), jnp.float32),
        pltpu.SemaphoreType.DMA,
        pltpu.SemaphoreType.DMA,
    ),
    compiler_params=pltpu.CompilerParams(needs_layout_passes=False))


def _dense_body(apply_act, split_out, aggL, aggR, cntp, w, b, *outs):
    nrows = aggL.shape[1]
    acc = jnp.zeros((nrows, D), jnp.float32)
    for r in range(R):
        cnt = cntp[r, :, 0]
        z = jax.lax.dot(aggL[r], w[r, :DH, :],
                        precision=jax.lax.Precision.HIGHEST,
                        preferred_element_type=jnp.float32)
        z = z + jax.lax.dot(aggR[r], w[r, DH:, :],
                            precision=jax.lax.Precision.HIGHEST,
                            preferred_element_type=jnp.float32)
        z = z + cnt[:, None] * b[r][None, :]
        acc = acc + z / jnp.maximum(cnt, 1.0)[:, None]
    if apply_act:
        acc = jnp.where(acc >= 0, acc, 0.01 * acc)
    if split_out:
        outs[0][0] = acc[:, :DH]
        outs[0][1] = acc[:, DH:]
    else:
        outs[0][...] = acc


def _dense_call(aggL, aggR, cntp, w, b, apply_act, split_out):
    blk = 2048
    grid = NPAD // blk
    if split_out:
        out_specs = pl.BlockSpec((NC, blk, DH), lambda i: (0, i, 0))
        out_shape = jax.ShapeDtypeStruct((NC, NPAD, DH), jnp.float32)
    else:
        out_specs = pl.BlockSpec((blk, D), lambda i: (i, 0))
        out_shape = jax.ShapeDtypeStruct((NPAD, D), jnp.float32)
    return pl.pallas_call(
        functools.partial(_dense_body, apply_act, split_out),
        grid=(grid,),
        in_specs=[
            pl.BlockSpec((R, blk, DH), lambda i: (0, i, 0)),
            pl.BlockSpec((R, blk, DH), lambda i: (0, i, 0)),
            pl.BlockSpec((R, blk, 16), lambda i: (0, i, 0)),
            pl.BlockSpec((R, D, D), lambda i: (0, 0, 0)),
            pl.BlockSpec((R, D), lambda i: (0, 0)),
        ],
        out_specs=out_specs,
        out_shape=out_shape,
    )(aggL, aggR, cntp, w, b)


def _pad_idx(vec, fill):
    return jnp.concatenate([vec, fill]).reshape(ROWS2D, C)


def kernel(x, W1, b1, W2, b2, ei0, ei1, ei2, nei0, nei1, nei2):
    pos = (ei0, ei1, ei2)
    npad = E_PAD - E
    src_fill = jnp.arange(npad, dtype=jnp.int32) % N
    dst_fill = N + (jnp.arange(npad, dtype=jnp.int32) % (NPAD - N))
    agg_idx = []
    for ei in pos:
        agg_idx.append(_pad_idx(ei[0], src_fill))
        agg_idx.append(_pad_idx(ei[1], dst_fill))

    agg1 = _make_agg_kernel(True)
    agg2 = _make_agg_kernel(False)

    xstk = jnp.stack([x[:, :DH], x[:, DH:]])
    aggL1, aggR1, cntp = agg1(xstk, *agg_idx)
    h1stk = _dense_call(aggL1, aggR1, cntp, W1, b1, True, True)
    aggL2, aggR2 = agg2(h1stk, *agg_idx)
    h2 = _dense_call(aggL2, aggR2, cntp, W2, b2, False, False)

    uall = jnp.concatenate(
        [_pad_idx(ei[0], src_fill)
         for ei in (ei0, ei1, ei2, nei0, nei1, nei2)])
    vall = jnp.concatenate(
        [_pad_idx(ei[1], src_fill)
         for ei in (ei0, ei1, ei2, nei0, nei1, nei2)])
    scores = _scores_kernel(h2, uall, vall).reshape(6, E_PAD)

    out_pos = scores[:R, :E].reshape(-1)
    out_neg = scores[R:, :E].reshape(-1)
    return (out_pos, out_neg)


# speedup vs baseline: 8.2188x; 1.1131x over previous
import functools

import jax
import jax.numpy as jnp
from jax import lax
from jax.experimental import pallas as pl
from jax.experimental.pallas import tpu as pltpu
from jax.experimental.pallas import tpu_sc as plsc

N = 10000
E = 160000
R = 3
D = 128
DH = D // 2

NC = 2
NS = 16
NW = NC * NS

C = 128
E_PAD = 163840
ROWS2D = E_PAD // C
KPT_AGG = ROWS2D // NS
KPT_SCO = ROWS2D // NW
EPT_SCO = C * KPT_SCO

NPAD = 10240
NSTRIPE = NPAD // NS

_mesh = plsc.VectorSubcoreMesh(
    core_axis_name="c", subcore_axis_name="s", num_cores=NC, num_subcores=NS)


def _fill_rows(ref, nrows, ncols, value):
    def body(r, _):
        for j in range(ncols // 16):
            ref[r, pl.ds(j * 16, 16)] = jnp.full((16,), value, jnp.float32)
        return 0
    lax.fori_loop(0, nrows, body, 0)


def _agg_body(with_counts, h_hbm, s0, d0, s1, d1, s2, d2, *rest):
    if with_counts:
        (aggL_hbm, aggR_hbm, cnt_hbm, sidx, didx, r0, r1, r2, r3, zbuf,
         zbuf16, ones, g0, g1, g2, g3, s0_, s1_, s2_, s3_, acc, cacc) = rest
    else:
        (aggL_hbm, aggR_hbm, sidx, didx, r0, r1, r2, r3, zbuf,
         g0, g1, g2, g3, s0_, s1_, s2_, s3_, acc) = rest
        cacc = cnt_hbm = zbuf16 = ones = None
    rows = (r0, r1, r2, r3)
    gsem = (g0, g1, g2, g3)
    ssem = (s0_, s1_, s2_, s3_)
    cid = lax.axis_index("c")
    sid = lax.axis_index("s")

    _fill_rows(zbuf, 128, DH, 0.0)
    if with_counts:
        _fill_rows(zbuf16, 128, 16, 0.0)
        _fill_ones = functools.partial(_fill_rows, ones, C, 16, 1.0)
        _fill_ones()

    srcs = (s0, s1, s2)
    dsts = (d0, d1, d2)
    for r in range(R):
        for j in range(5):
            pltpu.sync_copy(zbuf, acc.at[pl.ds(sid * NSTRIPE + j * 128, 128)])
            if with_counts:
                @pl.when(cid == 0)
                def _():
                    pltpu.sync_copy(
                        zbuf16, cacc.at[pl.ds(sid * NSTRIPE + j * 128, 128)])
        plsc.subcore_barrier()

        pltpu.sync_copy(srcs[r].at[pl.ds(sid * KPT_AGG, KPT_AGG)], sidx)
        pltpu.sync_copy(dsts[r].at[pl.ds(sid * KPT_AGG, KPT_AGG)], didx)

        NB = 4
        for b in range(NB):
            pltpu.async_copy(h_hbm.at[cid].at[sidx.at[b]], rows[b], gsem[b])

        def chunk_quad(k4, _):
            k = k4 * NB
            for b in range(NB):
                pltpu.make_async_copy(
                    h_hbm.at[cid].at[sidx.at[k + b]], rows[b], gsem[b]).wait()
                cp = pltpu.async_copy(rows[b], acc.at[didx.at[k + b]],
                                      ssem[b], add=True)
                if with_counts:
                    @pl.when(cid == 0)
                    def _():
                        pltpu.sync_copy(ones, cacc.at[didx.at[k + b]],
                                        add=True)
                cp.wait()

                @pl.when(k + b + NB < KPT_AGG)
                def _():
                    pltpu.async_copy(
                        h_hbm.at[cid].at[sidx.at[k + b + NB]], rows[b],
                        gsem[b])
            return 0
        lax.fori_loop(0, KPT_AGG // NB, chunk_quad, 0)
        plsc.subcore_barrier()

        @pl.when(cid == 0)
        def _():
            pltpu.sync_copy(acc.at[pl.ds(sid * NSTRIPE, NSTRIPE)],
                            aggL_hbm.at[r, pl.ds(sid * NSTRIPE, NSTRIPE)])
            if with_counts:
                pltpu.sync_copy(cacc.at[pl.ds(sid * NSTRIPE, NSTRIPE)],
                                cnt_hbm.at[r, pl.ds(sid * NSTRIPE, NSTRIPE)])

        @pl.when(cid == 1)
        def _():
            pltpu.sync_copy(acc.at[pl.ds(sid * NSTRIPE, NSTRIPE)],
                            aggR_hbm.at[r, pl.ds(sid * NSTRIPE, NSTRIPE)])
        plsc.subcore_barrier()


def _make_agg_kernel(with_counts):
    out_type = [jax.ShapeDtypeStruct((R, NPAD, DH), jnp.float32),
                jax.ShapeDtypeStruct((R, NPAD, DH), jnp.float32)]
    scratch = [
        pltpu.VMEM((KPT_AGG, C), jnp.int32),
        pltpu.VMEM((KPT_AGG, C), jnp.int32),
        pltpu.VMEM((C, DH), jnp.float32),
        pltpu.VMEM((C, DH), jnp.float32),
        pltpu.VMEM((C, DH), jnp.float32),
        pltpu.VMEM((C, DH), jnp.float32),
        pltpu.VMEM((128, DH), jnp.float32),
    ]
    if with_counts:
        out_type.append(jax.ShapeDtypeStruct((R, NPAD, 16), jnp.float32))
        scratch += [
            pltpu.VMEM((128, 16), jnp.float32),
            pltpu.VMEM((C, 16), jnp.float32),
        ]
    scratch += [pltpu.SemaphoreType.DMA] * 8 + [
                pltpu.VMEM_SHARED((NPAD, DH), jnp.float32)]
    if with_counts:
        scratch.append(pltpu.VMEM_SHARED((NPAD, 16), jnp.float32))
    return pl.kernel(functools.partial(_agg_body, with_counts),
                     out_type=tuple(out_type), mesh=_mesh,
                     scratch_types=tuple(scratch),
                     compiler_params=pltpu.CompilerParams(
                         use_tc_tiling_on_sc=False))


KPT_ALL = 6 * ROWS2D // NW


def _scores_body(h_hbm, uall, vall, out_hbm, *rest):
    (sidx, didx, hu0, hv0, hu1, hv1, hu2, hv2, tbuf, dots,
     gsem0, gsem1, gsem2) = rest
    cid = lax.axis_index("c")
    sid = lax.axis_index("s")
    wid = cid * NS + sid

    hu = (hu0, hu1, hu2)
    hv = (hv0, hv1, hv2)
    gsem = (gsem0, gsem1, gsem2)
    lane = lax.broadcasted_iota(jnp.int32, (16,), 0)
    KH = KPT_ALL // 4

    def issue(k, b):
        pltpu.async_copy(h_hbm.at[sidx.at[k]], hu[b], gsem[b])
        pltpu.async_copy(h_hbm.at[didx.at[k]], hv[b], gsem[b])

    def drain(k, b):
        pltpu.make_async_copy(h_hbm.at[sidx.at[k]], hu[b], gsem[b]).wait()
        pltpu.make_async_copy(h_hbm.at[didx.at[k]], hv[b], gsem[b]).wait()

    def compute(kout, b):
        def group(gi, _):
            e0 = gi * 16
            for i in range(16):
                e = e0 + i
                a = hu[b][e, pl.ds(0, 16)] * hv[b][e, pl.ds(0, 16)]
                for j in range(1, D // 16):
                    a = a + (hu[b][e, pl.ds(j * 16, 16)]
                             * hv[b][e, pl.ds(j * 16, 16)])
                tbuf[i, pl.ds(0, 16)] = a
            s = plsc.load_gather(tbuf, [lane, jnp.zeros((16,), jnp.int32)])
            for cc in range(1, 16):
                s = s + plsc.load_gather(
                    tbuf, [lane, jnp.full((16,), cc, jnp.int32)])
            s = 1.0 / (1.0 + jnp.exp(-s))
            dots[pl.ds(e0, 16)] = s
            return 0
        lax.fori_loop(0, C // 16, group, 0)
        off = (pl.multiple_of(wid * KPT_ALL * C, C)
               + pl.multiple_of(kout * C, C))
        pltpu.sync_copy(dots, out_hbm.at[pl.ds(off, C)])

    def phase(p, _):
        base = wid * KPT_ALL + p * KH
        pltpu.sync_copy(uall.at[pl.ds(base, KH)], sidx)
        pltpu.sync_copy(vall.at[pl.ds(base, KH)], didx)
        issue(0, 0)
        issue(1, 1)
        issue(2, 2)

        def chunk_tri(k3, _):
            k = k3 * 3
            for b in range(3):
                drain(k + b, b)
                compute(p * KH + k + b, b)

                @pl.when(k + b + 3 < KH)
                def _():
                    issue(k + b + 3, b)
            return 0
        lax.fori_loop(0, KH // 3, chunk_tri, 0)
        return 0
    lax.fori_loop(0, 4, phase, 0)


_scores_kernel = pl.kernel(
    _scores_body,
    out_type=jax.ShapeDtypeStruct((6 * E_PAD,), jnp.float32),
    mesh=_mesh,
    scratch_types=(
        pltpu.VMEM((KPT_ALL // 4, C), jnp.int32),
        pltpu.VMEM((KPT_ALL // 4, C), jnp.int32),
        pltpu.VMEM((C, D), jnp.float32),
        pltpu.VMEM((C, D), jnp.float32),
        pltpu.VMEM((C, D), jnp.float32),
        pltpu.VMEM((C, D), jnp.float32),
        pltpu.VMEM((C, D), jnp.float32),
        pltpu.VMEM((C, D), jnp.float32),
        pltpu.VMEM((16, 17), jnp.float32),
        pltpu.VMEM((C,), jnp.float32),
        pltpu.SemaphoreType.DMA,
        pltpu.SemaphoreType.DMA,
        pltpu.SemaphoreType.DMA,
    ),
    compiler_params=pltpu.CompilerParams(needs_layout_passes=False,
                                         use_tc_tiling_on_sc=False))


def _dense_body(apply_act, split_out, aggL, aggR, cntp, w, b, *outs):
    nrows = aggL.shape[1]
    acc = jnp.zeros((nrows, D), jnp.float32)
    for r in range(R):
        cnt = cntp[r, :, 0]
        z = jax.lax.dot(aggL[r], w[r, :DH, :],
                        precision=jax.lax.Precision.HIGHEST,
                        preferred_element_type=jnp.float32)
        z = z + jax.lax.dot(aggR[r], w[r, DH:, :],
                            precision=jax.lax.Precision.HIGHEST,
                            preferred_element_type=jnp.float32)
        z = z + cnt[:, None] * b[r][None, :]
        acc = acc + z / jnp.maximum(cnt, 1.0)[:, None]
    if apply_act:
        acc = jnp.where(acc >= 0, acc, 0.01 * acc)
    if split_out:
        outs[0][0] = acc[:, :DH]
        outs[0][1] = acc[:, DH:]
    else:
        outs[0][...] = acc


def _dense_call(aggL, aggR, cntp, w, b, apply_act, split_out):
    blk = 2048
    grid = NPAD // blk
    if split_out:
        out_specs = pl.BlockSpec((NC, blk, DH), lambda i: (0, i, 0))
        out_shape = jax.ShapeDtypeStruct((NC, NPAD, DH), jnp.float32)
    else:
        out_specs = pl.BlockSpec((blk, D), lambda i: (i, 0))
        out_shape = jax.ShapeDtypeStruct((NPAD, D), jnp.float32)
    return pl.pallas_call(
        functools.partial(_dense_body, apply_act, split_out),
        grid=(grid,),
        in_specs=[
            pl.BlockSpec((R, blk, DH), lambda i: (0, i, 0)),
            pl.BlockSpec((R, blk, DH), lambda i: (0, i, 0)),
            pl.BlockSpec((R, blk, 16), lambda i: (0, i, 0)),
            pl.BlockSpec((R, D, D), lambda i: (0, 0, 0)),
            pl.BlockSpec((R, D), lambda i: (0, 0)),
        ],
        out_specs=out_specs,
        out_shape=out_shape,
    )(aggL, aggR, cntp, w, b)


def _pad_idx(vec, fill):
    return jnp.concatenate([vec, fill]).reshape(ROWS2D, C)


def kernel(x, W1, b1, W2, b2, ei0, ei1, ei2, nei0, nei1, nei2):
    pos = (ei0, ei1, ei2)
    npad = E_PAD - E
    src_fill = jnp.arange(npad, dtype=jnp.int32) % N
    dst_fill = N + (jnp.arange(npad, dtype=jnp.int32) % (NPAD - N))
    agg_idx = []
    for ei in pos:
        agg_idx.append(_pad_idx(ei[0], src_fill))
        agg_idx.append(_pad_idx(ei[1], dst_fill))

    agg1 = _make_agg_kernel(True)
    agg2 = _make_agg_kernel(False)

    xstk = jnp.stack([x[:, :DH], x[:, DH:]])
    aggL1, aggR1, cntp = agg1(xstk, *agg_idx)
    h1stk = _dense_call(aggL1, aggR1, cntp, W1, b1, True, True)
    aggL2, aggR2 = agg2(h1stk, *agg_idx)
    h2 = _dense_call(aggL2, aggR2, cntp, W2, b2, False, False)

    uall = jnp.concatenate(
        [_pad_idx(ei[0], src_fill)
         for ei in (ei0, ei1, ei2, nei0, nei1, nei2)])
    vall = jnp.concatenate(
        [_pad_idx(ei[1], src_fill)
         for ei in (ei0, ei1, ei2, nei0, nei1, nei2)])
    scores = _scores_kernel(h2, uall, vall).reshape(6, E_PAD)

    out_pos = scores[:R, :E].reshape(-1)
    out_neg = scores[R:, :E].reshape(-1)
    return (out_pos, out_neg)
